# baseline (speedup 1.0000x reference)
import itertools

import jax
import jax.numpy as jnp
import numpy as np
from jax import lax
from jax.experimental import pallas as pl
from jax.experimental.pallas import tpu as pltpu
from jax.experimental.pallas import tpu_sc as plsc

NUM_BITS = 32
BATCH = 16384
NW = 32
ROWS_PER_W = BATCH // NW
BLK = 16
NBLK = ROWS_PER_W // BLK
NTERMS = 5488

_pairs = list(itertools.combinations(range(NUM_BITS), 2))
_trip_idx = {t: n for n, t in enumerate(itertools.combinations(range(NUM_BITS), 3))}
_PERM3 = np.array(
    [_trip_idx[(i, j, k)] for (j, k) in _pairs for i in range(j)], np.int32
)


def _pack_weights(w):
    w1 = w[:NUM_BITS]
    w2 = w[NUM_BITS:NUM_BITS + len(_pairs)]
    w3 = w[NUM_BITS + len(_pairs):]
    return jnp.concatenate([w1, w2, w3[_PERM3]])


def _sc_body(x_hbm, w_hbm, out_hbm, x_v, w_v, st_v, out_v):
    wid = lax.axis_index("s") * 2 + lax.axis_index("c")
    base = wid * ROWS_PER_W
    pltpu.sync_copy(x_hbm.at[pl.ds(base * NUM_BITS, ROWS_PER_W * NUM_BITS)], x_v)
    pltpu.sync_copy(w_hbm, w_v)
    lanes = lax.iota(jnp.int32, 16)

    def splat_w(idx):
        return plsc.load_gather(w_v, [jnp.full((16,), idx, jnp.int32)])

    def block_body(b, _):
        rb = b * BLK
        flat0 = rb * NUM_BITS + lanes * NUM_BITS

        def load_i(i, acc):
            g = plsc.load_gather(x_v, [flat0 + i])
            s = (1 - 2 * g).astype(jnp.float32)
            st_v[pl.ds(i * BLK, BLK)] = s
            return acc + splat_w(i) * s

        acc0 = lax.fori_loop(0, NUM_BITS, load_i, jnp.zeros((16,), jnp.float32))

        def j_body(j, c):
            accj, p2j, p3j = c
            sj = st_v[pl.ds(j * BLK, BLK)]

            def k_body(k, c2):
                acck, p2, p3 = c2
                sk = st_v[pl.ds(k * BLK, BLK)]

                def i_body(i, inner):
                    return inner + splat_w(p3 + i) * st_v[pl.ds(i * BLK, BLK)]

                inner = lax.fori_loop(0, j, i_body, splat_w(p2))
                return (acck + sj * sk * inner, p2 + 1, p3 + j)

            return lax.fori_loop(j + 1, NUM_BITS, k_body, (accj, p2j, p3j))

        acc, _, _ = lax.fori_loop(
            0, NUM_BITS, j_body,
            (acc0, jnp.int32(NUM_BITS), jnp.int32(NUM_BITS + len(_pairs))),
        )
        out_v[pl.ds(rb, BLK)] = acc
        return 0

    lax.fori_loop(0, NBLK, block_body, 0)
    pltpu.sync_copy(out_v, out_hbm.at[pl.ds(base, ROWS_PER_W)])


@jax.jit
def kernel(inputs, kernel):
    w_all = _pack_weights(kernel)
    run = pl.kernel(
        _sc_body,
        out_type=jax.ShapeDtypeStruct((BATCH,), jnp.float32),
        mesh=plsc.VectorSubcoreMesh(core_axis_name="c", subcore_axis_name="s"),
        compiler_params=pltpu.CompilerParams(needs_layout_passes=False),
        scratch_types=[
            pltpu.VMEM((ROWS_PER_W * NUM_BITS,), jnp.int32),
            pltpu.VMEM((NTERMS,), jnp.float32),
            pltpu.VMEM((NUM_BITS * BLK,), jnp.float32),
            pltpu.VMEM((ROWS_PER_W,), jnp.float32),
        ],
    )
    return run(inputs.reshape(BATCH * NUM_BITS), w_all)


# speedup vs baseline: 11.2115x; 11.2115x over previous
import itertools

import jax
import jax.numpy as jnp
import numpy as np
from jax import lax
from jax.experimental import pallas as pl
from jax.experimental.pallas import tpu as pltpu
from jax.experimental.pallas import tpu_sc as plsc

NUM_BITS = 32
BATCH = 16384
NTERMS = 5488

_N1 = 32
_PAIRS = list(itertools.combinations(range(NUM_BITS), 2))
_N2 = len(_PAIRS)
_C3 = np.array(list(itertools.combinations(range(NUM_BITS), 3)), np.int32)
_N3 = _C3.shape[0]

SC_ROWS = 1024
TC_ROWS = BATCH - SC_ROWS
NW = 32
ROWS_PER_W = SC_ROWS // NW
BLK = 16
NBLK = ROWS_PER_W // BLK


_NCOL = 640

_PAIR_COL = {(int(j), int(k)): p for p, (j, k) in enumerate(_PAIRS)}

_EJ = np.zeros((NUM_BITS, _NCOL), np.float32)
_EK = np.zeros((NUM_BITS, _NCOL), np.float32)
for p, (j, k) in enumerate(_PAIRS):
    _EJ[j, p] = 1.0
    _EK[k, p] = 1.0
for i in range(_N1):
    _EJ[i, _N2 + i] = 1.0
    _EK[i, _N2 + i] = 1.0

_W3_ROWS = _C3[:, 0]
_W3_COLS = np.array([_PAIR_COL[(int(j), int(k))] for (_, j, k) in _C3], np.int32)
_W1_ROWS = np.arange(_N1, dtype=np.int32)
_W1_COLS = np.arange(_N2, _N2 + _N1, dtype=np.int32)


def _build_tc_constants(w):
    w1, w2, w3 = w[:_N1], w[_N1:_N1 + _N2], w[_N1 + _N2:]
    w3mat = jnp.zeros((NUM_BITS, _NCOL), jnp.float32)
    w3mat = w3mat.at[_W3_ROWS, _W3_COLS].set(w3)
    w3mat = w3mat.at[_W1_ROWS, _W1_COLS].set(w1)
    w2pad = jnp.zeros((1, _NCOL), jnp.float32).at[0, :_N2].set(w2)
    cm = jnp.concatenate([jnp.asarray(_EJ), jnp.asarray(_EK), w3mat], axis=1)
    return cm, w2pad


_BBLK = 1024


def _tc_body(x_ref, cm_ref, w2_ref, out_ref):
    s = (1 - 2 * x_ref[...]).astype(jnp.float32)
    prod = jax.lax.dot_general(
        s, cm_ref[...], (((1,), (0,)), ((), ())),
        preferred_element_type=jnp.float32,
    )
    sj = prod[:, :_NCOL]
    sk = prod[:, _NCOL:2 * _NCOL]
    a2 = prod[:, 2 * _NCOL:] + w2_ref[...]
    out_ref[...] = jnp.sum(sj * sk * a2, axis=1, keepdims=True)


def _tc_half(x, cm, w2pad):
    grid = TC_ROWS // _BBLK
    out = pl.pallas_call(
        _tc_body,
        grid=(grid,),
        in_specs=[
            pl.BlockSpec((_BBLK, NUM_BITS), lambda i: (i, 0)),
            pl.BlockSpec((NUM_BITS, 3 * _NCOL), lambda i: (0, 0)),
            pl.BlockSpec((1, _NCOL), lambda i: (0, 0)),
        ],
        out_specs=pl.BlockSpec((_BBLK, 1), lambda i: (i, 0)),
        out_shape=jax.ShapeDtypeStruct((TC_ROWS, 1), jnp.float32),
        compiler_params=pltpu.CompilerParams(
            dimension_semantics=("arbitrary",),
        ),
    )(x, cm, w2pad)
    return out[:, 0]


_trip_idx = {t: n for n, t in enumerate(itertools.combinations(range(NUM_BITS), 3))}
_PERM3 = np.array(
    [_trip_idx[(i, j, k)] for (j, k) in _PAIRS for i in range(j)], np.int32
)


def _pack_sc_weights(w):
    w1, w2, w3 = w[:_N1], w[_N1:_N1 + _N2], w[_N1 + _N2:]
    return jnp.concatenate([w1, w2, w3[_PERM3]])


def _sc_body(x_hbm, w_hbm, out_hbm, x_v, w_v, st_v, out_v):
    wid = lax.axis_index("s") * 2 + lax.axis_index("c")
    base = wid * ROWS_PER_W
    pltpu.sync_copy(x_hbm.at[pl.ds(base * NUM_BITS, ROWS_PER_W * NUM_BITS)], x_v)
    pltpu.sync_copy(w_hbm, w_v)
    lanes = lax.iota(jnp.int32, 16)

    def splat_w(idx):
        return plsc.load_gather(w_v, [jnp.full((16,), idx, jnp.int32)])

    def block_body(b, _):
        rb = b * BLK
        flat0 = rb * NUM_BITS + lanes * NUM_BITS

        def load_i(i, acc):
            g = plsc.load_gather(x_v, [flat0 + i])
            s = (1 - 2 * g).astype(jnp.float32)
            st_v[pl.ds(i * BLK, BLK)] = s
            return acc + splat_w(i) * s

        acc0 = lax.fori_loop(0, NUM_BITS, load_i, jnp.zeros((16,), jnp.float32))

        def j_body(j, c):
            accj, p2j, p3j = c
            sj = st_v[pl.ds(j * BLK, BLK)]

            def k_body(k, c2):
                acck, p2, p3 = c2
                sk = st_v[pl.ds(k * BLK, BLK)]

                def i_body(i, inner):
                    return inner + splat_w(p3 + i) * st_v[pl.ds(i * BLK, BLK)]

                inner = lax.fori_loop(0, j, i_body, splat_w(p2))
                return (acck + sj * sk * inner, p2 + 1, p3 + j)

            return lax.fori_loop(j + 1, NUM_BITS, k_body, (accj, p2j, p3j))

        acc, _, _ = lax.fori_loop(
            0, NUM_BITS, j_body,
            (acc0, jnp.int32(_N1), jnp.int32(_N1 + _N2)),
        )
        out_v[pl.ds(rb, BLK)] = acc
        return 0

    lax.fori_loop(0, NBLK, block_body, 0)
    pltpu.sync_copy(out_v, out_hbm.at[pl.ds(base, ROWS_PER_W)])


def _sc_half(x_flat, w_all):
    run = pl.kernel(
        _sc_body,
        out_type=jax.ShapeDtypeStruct((SC_ROWS,), jnp.float32),
        mesh=plsc.VectorSubcoreMesh(core_axis_name="c", subcore_axis_name="s"),
        compiler_params=pltpu.CompilerParams(needs_layout_passes=False),
        scratch_types=[
            pltpu.VMEM((ROWS_PER_W * NUM_BITS,), jnp.int32),
            pltpu.VMEM((NTERMS,), jnp.float32),
            pltpu.VMEM((NUM_BITS * BLK,), jnp.float32),
            pltpu.VMEM((ROWS_PER_W,), jnp.float32),
        ],
    )
    return run(x_flat, w_all)


@jax.jit
def kernel(inputs, kernel):
    cm, w2pad = _build_tc_constants(kernel)
    w_all = _pack_sc_weights(kernel)
    e_sc = _sc_half(inputs[TC_ROWS:].reshape(SC_ROWS * NUM_BITS), w_all)
    e_tc = _tc_half(inputs[:TC_ROWS], cm, w2pad)
    return jnp.concatenate([e_tc, e_sc])
